# baseline (speedup 1.0000x reference)
import functools
import math

import jax
import jax.numpy as jnp
from jax import lax
from jax.experimental import pallas as pl
from jax.experimental.pallas import tpu as pltpu
from jax.experimental.pallas import tpu_sc as plsc

D_MODEL = 128
SCALE = math.sqrt(D_MODEL)
LANES = 16

NUM_CORES = 2
NUM_SUBCORES = 16
NUM_WORKERS = NUM_CORES * NUM_SUBCORES


@functools.lru_cache(maxsize=None)
def _make_kernel(B: int, D: int, C: int):
    assert B % NUM_WORKERS == 0
    per_w = B // NUM_WORKERS
    assert per_w % C == 0
    n_chunks = per_w // C
    assert C % 8 == 0 and D % LANES == 0

    mesh = plsc.VectorSubcoreMesh(core_axis_name="c", subcore_axis_name="s")

    NBUF = 3
    DEPTH = 1
    NSPLIT = 2
    assert (C // NSPLIT) % 8 == 0

    @functools.partial(
        pl.kernel,
        mesh=mesh,
        out_type=jax.ShapeDtypeStruct((B, D), jnp.float32),
        scratch_types=[
            pltpu.VMEM((per_w,), jnp.int32),
        ] + [pltpu.VMEM((C, D), jnp.float32)] * NBUF
          + [pltpu.SemaphoreType.DMA] * (2 * NBUF),
    )
    def emb_kernel(table_hbm, idx_hbm, out_hbm, idx_v, *scratch):
        bufs = scratch[:NBUF]
        sems_in = scratch[NBUF:2 * NBUF]
        sems_out = scratch[2 * NBUF:]
        wid = lax.axis_index("s") * NUM_CORES + lax.axis_index("c")
        base = wid * per_w
        pltpu.sync_copy(idx_hbm.at[pl.ds(base, per_w)], idx_v)

        def gather(g):
            b = g % NBUF
            h = C // NSPLIT
            return [
                pltpu.async_copy(
                    table_hbm.at[idx_v.at[pl.ds(g * C + s * h, h)]],
                    bufs[b].at[pl.ds(s * h, h)], sems_in[b])
                for s in range(NSPLIT)
            ]

        def scale(buf):
            def row_body(r, carry):
                for j in range(D // LANES):
                    sl = pl.ds(j * LANES, LANES)
                    buf[r, sl] = buf[r, sl] * SCALE
                return carry
            lax.fori_loop(0, C, row_body, 0, unroll=2)

        gathers = [None] * n_chunks
        outs = [None] * n_chunks
        for g in range(DEPTH):
            gathers[g] = gather(g)
        for g in range(n_chunks):
            b = g % NBUF
            if g + DEPTH < n_chunks:
                prev = g + DEPTH - NBUF
                if prev >= 0:
                    outs[prev].wait()
                gathers[g + DEPTH] = gather(g + DEPTH)
            for cp in gathers[g]:
                cp.wait()
            scale(bufs[b])
            outs[g] = pltpu.async_copy(
                bufs[b], out_hbm.at[pl.ds(base + g * C, C)], sems_out[b])
        for g in range(max(0, n_chunks - NBUF), n_chunks):
            outs[g].wait()

    return emb_kernel


def kernel(inputs, table):
    B = inputs.shape[0] * inputs.shape[1]
    D = table.shape[1]
    idx_flat = inputs.reshape(B).astype(jnp.int32)
    out = _make_kernel(B, D, 320)(table, idx_flat)
    return out.reshape(inputs.shape[0], inputs.shape[1], D)


# speedup vs baseline: 1.0107x; 1.0107x over previous
import functools
import math

import jax
import jax.numpy as jnp
from jax import lax
from jax.experimental import pallas as pl
from jax.experimental.pallas import tpu as pltpu
from jax.experimental.pallas import tpu_sc as plsc

D_MODEL = 128
SCALE = math.sqrt(D_MODEL)
LANES = 16

NUM_CORES = 2
NUM_SUBCORES = 16
NUM_WORKERS = NUM_CORES * NUM_SUBCORES


@functools.lru_cache(maxsize=None)
def _make_kernel(B: int, D: int, C: int):
    assert B % NUM_WORKERS == 0
    per_w = B // NUM_WORKERS
    assert per_w % C == 0
    n_chunks = per_w // C
    assert C % 8 == 0 and D % LANES == 0

    mesh = plsc.VectorSubcoreMesh(core_axis_name="c", subcore_axis_name="s")

    NBUF = 4
    DEPTH = 2

    @functools.partial(
        pl.kernel,
        mesh=mesh,
        out_type=jax.ShapeDtypeStruct((B, D), jnp.float32),
        scratch_types=[
            pltpu.VMEM((per_w,), jnp.int32),
        ] + [pltpu.VMEM((C, D), jnp.float32)] * NBUF
          + [pltpu.SemaphoreType.DMA] * (2 * NBUF),
    )
    def emb_kernel(table_hbm, idx_hbm, out_hbm, idx_v, *scratch):
        bufs = scratch[:NBUF]
        sems_in = scratch[NBUF:2 * NBUF]
        sems_out = scratch[2 * NBUF:]
        wid = lax.axis_index("s") * NUM_CORES + lax.axis_index("c")
        base = wid * per_w

        def gather(g):
            b = g % NBUF
            return pltpu.async_copy(
                table_hbm.at[idx_v.at[pl.ds(g * C, C)]], bufs[b], sems_in[b])

        def scale(buf):
            def row_body(r, carry):
                for j in range(D // LANES):
                    sl = pl.ds(j * LANES, LANES)
                    buf[r, sl] = buf[r, sl] * SCALE
                return carry
            lax.fori_loop(0, C, row_body, 0, unroll=2)

        idx_head = pltpu.async_copy(
            idx_hbm.at[pl.ds(base, C)], idx_v.at[pl.ds(0, C)], sems_in[0])
        idx_rest = pltpu.async_copy(
            idx_hbm.at[pl.ds(base + C, per_w - C)],
            idx_v.at[pl.ds(C, per_w - C)], sems_in[1])

        gathers = [None] * n_chunks
        outs = [None] * n_chunks
        idx_head.wait()
        gathers[0] = gather(0)
        idx_rest.wait()
        for g in range(1, DEPTH):
            gathers[g] = gather(g)
        for g in range(n_chunks):
            b = g % NBUF
            if g + DEPTH < n_chunks:
                prev = g + DEPTH - NBUF
                if prev >= 0:
                    outs[prev].wait()
                gathers[g + DEPTH] = gather(g + DEPTH)
            gathers[g].wait()
            scale(bufs[b])
            outs[g] = pltpu.async_copy(
                bufs[b], out_hbm.at[pl.ds(base + g * C, C)], sems_out[b])
        for g in range(max(0, n_chunks - NBUF), n_chunks):
            outs[g].wait()

    return emb_kernel


def kernel(inputs, table):
    B = inputs.shape[0] * inputs.shape[1]
    D = table.shape[1]
    idx_flat = inputs.reshape(B).astype(jnp.int32)
    out = _make_kernel(B, D, 200)(table, idx_flat)
    return out.reshape(inputs.shape[0], inputs.shape[1], D)
